# Initial kernel scaffold; baseline (speedup 1.0000x reference)
import jax
import jax.numpy as jnp
from jax import lax
from jax.experimental import pallas as pl
from jax.experimental.pallas import tpu as pltpu
from jax.experimental.pallas import tpu_sc as plsc

NF = 26
V = 100000
B = 16384
NC, NS, L = 2, 16, 16
NW = NC * NS
BPW = B // NW
EPW = BPW * NF
ROWS = EPW // 128


def _body(x2d_hbm, table_hbm, out_hbm, idx_v, vals_v, out_v, sem):
    wid = lax.axis_index("s") * NC + lax.axis_index("c")
    iota = lax.iota(jnp.int32, L)

    pltpu.sync_copy(x2d_hbm.at[pl.ds(wid * ROWS, ROWS)], idx_v)

    def fix_row(r, _):
        for j in range(8):
            p0 = r * 128 + j * 16
            f = lax.rem(iota + p0, NF)
            idx_v[r, pl.ds(j * 16, 16)] = idx_v[r, pl.ds(j * 16, 16)] + f * V
        return 0
    lax.fori_loop(0, ROWS, fix_row, 0)

    pltpu.async_copy(table_hbm.at[idx_v], vals_v, sem).wait()

    base26 = iota * NF
    def red_chunk(c, _):
        acc = jnp.zeros((L,), jnp.float32)
        for f in range(NF):
            p = base26 + (c * (L * NF) + f)
            acc = acc + plsc.load_gather(
                vals_v, [lax.shift_right_logical(p, 7), lax.bitwise_and(p, 127)])
        out_v[pl.ds(c * L, L)] = acc
        return 0
    lax.fori_loop(0, BPW // L, red_chunk, 0)

    pltpu.sync_copy(out_v, out_hbm.at[pl.ds(wid * BPW, BPW)])


def kernel(X, table):
    x2d = X.reshape(NW * ROWS, 128)
    tflat = table.reshape(NF * V)
    mesh = plsc.VectorSubcoreMesh(core_axis_name="c", subcore_axis_name="s")
    out = pl.kernel(
        _body,
        out_type=jax.ShapeDtypeStruct((B,), jnp.float32),
        mesh=mesh,
        scratch_types=[
            pltpu.VMEM((ROWS, 128), jnp.int32),
            pltpu.VMEM((ROWS, 128), jnp.float32),
            pltpu.VMEM((BPW,), jnp.float32),
            pltpu.SemaphoreType.DMA,
        ],
    )(x2d, tflat)
    return out.reshape(B, 1)


# speedup vs baseline: 1.4021x; 1.4021x over previous
import jax
import jax.numpy as jnp
from jax import lax
from jax.experimental import pallas as pl
from jax.experimental.pallas import tpu as pltpu
from jax.experimental.pallas import tpu_sc as plsc

NF = 26
V = 100000
B = 16384
NC, NS, L = 2, 16, 16
NW = NC * NS
CHUNK = 4096
BPW = B // NW

_CP = pltpu.CompilerParams(needs_layout_passes=False)
_MESH = dict(core_axis_name="c", subcore_axis_name="s",
             num_cores=NC, num_subcores=NS)


def _gather_body(xt_hbm, table_hbm, part_hbm, trow_v, idx_v, val_v, sem):
    wid = lax.axis_index("s") * NC + lax.axis_index("c")

    @pl.when(wid < NF)
    def _():
        f = wid
        pltpu.sync_copy(table_hbm.at[f], trow_v)

        def chunk_step(c, _):
            base = c * CHUNK
            pltpu.sync_copy(xt_hbm.at[f, pl.ds(base, CHUNK)], idx_v)

            def vec_step(i, _):
                ids = idx_v[pl.ds(i * L, L)]
                val_v[pl.ds(i * L, L)] = plsc.load_gather(trow_v, [ids])
                return 0
            lax.fori_loop(0, CHUNK // L, vec_step, 0)
            pltpu.sync_copy(val_v, part_hbm.at[f, pl.ds(base, CHUNK)])
            return 0
        lax.fori_loop(0, B // CHUNK, chunk_step, 0)


def _reduce_body(part_hbm, out_hbm, pv, out_v):
    wid = lax.axis_index("s") * NC + lax.axis_index("c")
    pltpu.sync_copy(part_hbm.at[:, pl.ds(wid * BPW, BPW)], pv)

    def red_chunk(c, _):
        acc = jnp.zeros((L,), jnp.float32)
        for f in range(NF):
            acc = acc + pv[f, pl.ds(c * L, L)]
        out_v[pl.ds(c * L, L)] = acc
        return 0
    lax.fori_loop(0, BPW // L, red_chunk, 0)
    pltpu.sync_copy(out_v, out_hbm.at[pl.ds(wid * BPW, BPW)])


def kernel(X, table):
    xt = X.T
    t2d = table.reshape(NF, V)

    mesh1 = plsc.VectorSubcoreMesh(**_MESH)
    partials = pl.kernel(
        _gather_body,
        out_type=jax.ShapeDtypeStruct((NF, B), jnp.float32),
        mesh=mesh1,
        scratch_types=[
            pltpu.VMEM((V,), jnp.float32),
            pltpu.VMEM((CHUNK,), jnp.int32),
            pltpu.VMEM((CHUNK,), jnp.float32),
            pltpu.SemaphoreType.DMA,
        ],
        compiler_params=_CP,
    )(xt, t2d)

    mesh2 = plsc.VectorSubcoreMesh(**_MESH)
    out = pl.kernel(
        _reduce_body,
        out_type=jax.ShapeDtypeStruct((B,), jnp.float32),
        mesh=mesh2,
        scratch_types=[
            pltpu.VMEM((NF, BPW), jnp.float32),
            pltpu.VMEM((BPW,), jnp.float32),
        ],
        compiler_params=_CP,
    )(partials)
    return out.reshape(B, 1)
